# baseline (speedup 1.0000x reference)
import functools
import jax
import jax.numpy as jnp
from jax import lax
from jax.experimental import pallas as pl
from jax.experimental.pallas import tpu as pltpu, tpu_sc as plsc

_INTERPRET = False

IN_CH = 3
HID = 256
INP = 128
LS = 32
N32 = LS * LS


def _mm(a, b):
    return jax.lax.dot_general(a, b, (((1,), (0,)), ((), ())),
                               preferred_element_type=jnp.float32)


def _mmb(a, b):
    return jax.lax.dot_general(a.astype(jnp.bfloat16), b.astype(jnp.bfloat16),
                               (((1,), (0,)), ((), ())),
                               preferred_element_type=jnp.float32)


def _conv3x3(hflat, T, b, mm=_mm):
    hp = jnp.pad(hflat.reshape(LS, LS, HID), ((1, 1), (1, 1), (0, 0)))
    a = None
    for dy in range(3):
        for dx in range(3):
            sl = hp[dy:dy + LS, dx:dx + LS, :].reshape(N32, HID)
            t = mm(sl, T[dy * 3 + dx])
            a = t if a is None else a + t
    return a + b


def _resblock(hflat, T, bT, W, bW, mm=_mm):
    r = jax.nn.relu(hflat)
    r = jax.nn.relu(_conv3x3(r, T, bT, mm))
    return hflat + mm(r, W) + bW


def _enc_body(p1_r, w1m_r, b1_r, t2_r, b2_r, t3_r, b3_r,
              tr3a_r, br3a_r, wr1a_r, br1a_r,
              tr3b_r, br3b_r, wr1b_r, br1b_r, cbm_r,
              z_o, idx_o):
    relu = jax.nn.relu
    h1 = relu(_mm(p1_r[...], w1m_r[...]) + b1_r[...])
    h1p = [[None, None], [None, None]]
    for q in (0, 1):
        for qx in (0, 1):
            blk = h1[(q * 2 + qx) * N32:(q * 2 + qx + 1) * N32, :]
            h1p[q][qx] = jnp.pad(blk.reshape(LS, LS, HID // 2),
                                 ((1, 1), (1, 1), (0, 0)))
    amap = {0: (1, 0), 1: (0, 1), 2: (1, 1), 3: (0, 2)}
    t2 = t2_r[...]
    acc = None
    for dy in range(4):
        q, a0 = amap[dy]
        for dx in range(4):
            qx, b0 = amap[dx]
            sl = h1p[q][qx][a0:a0 + LS, b0:b0 + LS, :].reshape(N32, HID // 2)
            t = _mm(sl, t2[dy * 4 + dx])
            acc = t if acc is None else acc + t
    h = relu(acc + b2_r[...])
    h = _conv3x3(h, t3_r[...], b3_r[...])
    h = _resblock(h, tr3a_r[...], br3a_r[...], wr1a_r[...], br1a_r[...])
    h = _resblock(h, tr3b_r[...], br3b_r[...], wr1b_r[...], br1b_r[...])
    z = h
    z_o[...] = z

    cbm = cbm_r[...]
    c2 = jnp.sum(cbm * cbm, axis=0, keepdims=True)
    dist = c2 - 2.0 * _mm(z, cbm)
    mn = jnp.min(dist, axis=1, keepdims=True)
    ii = jax.lax.broadcasted_iota(jnp.int32, (N32, N32), 1)
    idx_o[...] = jnp.min(jnp.where(dist == mn, ii, jnp.int32(1 << 30)),
                         axis=1, keepdims=True)


def _dec_body(zq_r, td0_r, bd0_r,
              dr3a_r, dbr3a_r, dw1a_r, dbr1a_r,
              dr3b_r, dbr3b_r, dw1b_r, dbr1b_r,
              tt1_r, tb1_r, tt2_r, tb2_r, xp_o):
    relu = jax.nn.relu
    zq = zq_r[...]
    h = _conv3x3(zq, td0_r[...], bd0_r[...], _mmb)
    h = _resblock(h, dr3a_r[...], dbr3a_r[...], dw1a_r[...], dbr1a_r[...], _mmb)
    h = _resblock(h, dr3b_r[...], dbr3b_r[...], dw1b_r[...], dbr1b_r[...], _mmb)

    hp = jnp.pad(h.reshape(LS, LS, HID), ((1, 1), (1, 1), (0, 0)))
    tt1 = tt1_r[...]
    G = [[None, None], [None, None]]
    for s in (0, 1):
        for sx in (0, 1):
            a = None
            for dy in (s, s + 2):
                a0 = (s - 2 + dy) // 2 + 1
                for dx in (sx, sx + 2):
                    b0 = (sx - 2 + dx) // 2 + 1
                    sl = hp[a0:a0 + LS, b0:b0 + LS, :].reshape(N32, HID)
                    t = _mmb(sl, tt1[dy * 4 + dx])
                    a = t if a is None else a + t
            g = relu(a + tb1_r[...])
            G[s][sx] = jnp.pad(g.reshape(LS, LS, HID // 2),
                               ((1, 1), (1, 1), (0, 0)))
    tt2 = tt2_r[...]
    outs = []
    for ry in range(4):
        for rx in range(4):
            a = None
            for dy in (ry % 2, ry % 2 + 2):
                m = (ry - 2 + dy) // 2
                s = m % 2
                a0 = 1 + (m - s) // 2
                for dx in (rx % 2, rx % 2 + 2):
                    mx = (rx - 2 + dx) // 2
                    sx = mx % 2
                    b0 = 1 + (mx - sx) // 2
                    sl = G[s][sx][a0:a0 + LS, b0:b0 + LS, :].reshape(N32, HID // 2)
                    t = _mmb(sl, tt2[dy * 4 + dx])
                    a = t if a is None else a + t
            outs.append(a + tb2_r[...])
    xp_o[...] = jnp.concatenate(outs, axis=1)


def _sc_gather(table, idx):
    info = plsc.get_sparse_core_info()
    NC, NS = info.num_cores, info.num_subcores
    NW = NC * NS
    b_per_w = N32 // NW
    mesh = plsc.VectorSubcoreMesh(core_axis_name="c", subcore_axis_name="s")

    @functools.partial(
        pl.kernel, mesh=mesh,
        out_type=jax.ShapeDtypeStruct((N32, HID), jnp.float32),
        scratch_types=[
            pltpu.VMEM((b_per_w,), jnp.int32),
            pltpu.VMEM((b_per_w, HID), jnp.float32),
            pltpu.SemaphoreType.DMA,
        ],
    )
    def k(table_hbm, idx_hbm, out_hbm, idx_v, rows_v, sem):
        wid = lax.axis_index("s") * NC + lax.axis_index("c")
        base = wid * b_per_w
        pltpu.sync_copy(idx_hbm.at[pl.ds(base, b_per_w)], idx_v)
        pltpu.async_copy(table_hbm.at[idx_v], rows_v, sem).wait()
        pltpu.sync_copy(rows_v, out_hbm.at[pl.ds(base, b_per_w)])

    return k(table, idx)


def kernel(x, code_books, params):
    p = params
    f32 = jnp.float32

    x_cl = x.transpose(1, 2, 0)
    xpad = jnp.pad(x_cl, ((1, 1), (1, 1), (0, 0)))
    span = 4 * (LS - 1) + 2
    blocks = []
    for q in (0, 1):
        for qx in (0, 1):
            patch = jnp.concatenate(
                [xpad[2 * q + dy:2 * q + dy + span:4,
                      2 * qx + dx:2 * qx + dx + span:4, :]
                 for dy in range(4) for dx in range(4)], axis=-1)
            blocks.append(patch.reshape(N32, 48))
    p1 = jnp.concatenate(blocks, axis=0)

    def taps(w, k):
        return w.transpose(2, 3, 1, 0).reshape(k * k, w.shape[1], w.shape[0])

    w1m = p['enc_w1'].transpose(2, 3, 1, 0).reshape(48, HID // 2)
    row = lambda b: b.reshape(1, -1)

    enc_args = [
        p1, w1m, row(p['enc_b1']),
        taps(p['enc_w2'], 4), row(p['enc_b2']),
        taps(p['enc_w3'], 3), row(p['enc_b3']),
        taps(p['enc_res_w3_0'], 3), row(p['enc_res_b3_0']),
        p['enc_res_w1_0'][:, :, 0, 0].T, row(p['enc_res_b1_0']),
        taps(p['enc_res_w3_1'], 3), row(p['enc_res_b3_1']),
        p['enc_res_w1_1'][:, :, 0, 0].T, row(p['enc_res_b1_1']),
        code_books.reshape(HID, N32),
    ]
    zf, idx2d = pl.pallas_call(
        _enc_body,
        out_shape=(jax.ShapeDtypeStruct((N32, HID), f32),
                   jax.ShapeDtypeStruct((N32, 1), jnp.int32)),
        interpret=_INTERPRET,
    )(*enc_args)

    cbt = code_books.reshape(HID, N32).T
    zqf = _sc_gather(cbt, idx2d.reshape(N32))

    dec_args = [
        zqf,
        taps(p['dec_w0'], 3), row(p['dec_b0']),
        taps(p['dec_res_w3_0'], 3), row(p['dec_res_b3_0']),
        p['dec_res_w1_0'][:, :, 0, 0].T, row(p['dec_res_b1_0']),
        taps(p['dec_res_w3_1'], 3), row(p['dec_res_b3_1']),
        p['dec_res_w1_1'][:, :, 0, 0].T, row(p['dec_res_b1_1']),
        taps(p['dec_tw1'], 4), row(p['dec_tb1']),
        taps(p['dec_tw2'], 4), row(p['dec_tb2']),
    ]
    xp = pl.pallas_call(
        _dec_body,
        out_shape=jax.ShapeDtypeStruct((N32, 48), f32),
        interpret=_INTERPRET,
    )(*dec_args)

    x_pred = (xp.reshape(LS, LS, 4, 4, 3)
                .transpose(4, 0, 2, 1, 3).reshape(3, INP, INP))
    z_st = zf.reshape(LS, LS, HID).transpose(2, 0, 1)
    zq = zqf.reshape(LS, LS, HID).transpose(2, 0, 1)
    return (x_pred, z_st, zq)


# speedup vs baseline: 1.0053x; 1.0053x over previous
import functools
import jax
import jax.numpy as jnp
from jax import lax
from jax.experimental import pallas as pl
from jax.experimental.pallas import tpu as pltpu, tpu_sc as plsc

IN_CH = 3
HID = 256
INP = 128
LS = 32
N32 = LS * LS


def _mm(a, b):
    return jax.lax.dot_general(a, b, (((1,), (0,)), ((), ())),
                               preferred_element_type=jnp.float32)


def _mmb(a, b):
    return jax.lax.dot_general(a.astype(jnp.bfloat16), b.astype(jnp.bfloat16),
                               (((1,), (0,)), ((), ())),
                               preferred_element_type=jnp.float32)


def _conv3x3(hflat, T, b, mm=_mm):
    hp = jnp.pad(hflat.reshape(LS, LS, HID), ((1, 1), (1, 1), (0, 0)))
    a = None
    for dy in range(3):
        for dx in range(3):
            sl = hp[dy:dy + LS, dx:dx + LS, :].reshape(N32, HID)
            t = mm(sl, T[dy * 3 + dx])
            a = t if a is None else a + t
    return a + b


def _resblock(hflat, T, bT, W, bW, mm=_mm):
    r = jax.nn.relu(hflat)
    r = jax.nn.relu(_conv3x3(r, T, bT, mm))
    return hflat + mm(r, W) + bW


def _enc_body(p1_r, w1m_r, b1_r, t2_r, b2_r, t3_r, b3_r,
              tr3a_r, br3a_r, wr1a_r, br1a_r,
              tr3b_r, br3b_r, wr1b_r, br1b_r, cbm_r,
              z_o, idx_o):
    relu = jax.nn.relu
    h1 = relu(_mm(p1_r[...], w1m_r[...]) + b1_r[...])
    h1p = [[None, None], [None, None]]
    for q in (0, 1):
        for qx in (0, 1):
            blk = h1[(q * 2 + qx) * N32:(q * 2 + qx + 1) * N32, :]
            h1p[q][qx] = jnp.pad(blk.reshape(LS, LS, HID // 2),
                                 ((1, 1), (1, 1), (0, 0)))
    amap = {0: (1, 0), 1: (0, 1), 2: (1, 1), 3: (0, 2)}
    t2 = t2_r[...]
    acc = None
    for dy in range(4):
        q, a0 = amap[dy]
        for dx in range(4):
            qx, b0 = amap[dx]
            sl = h1p[q][qx][a0:a0 + LS, b0:b0 + LS, :].reshape(N32, HID // 2)
            t = _mm(sl, t2[dy * 4 + dx])
            acc = t if acc is None else acc + t
    h = relu(acc + b2_r[...])
    h = _conv3x3(h, t3_r[...], b3_r[...])
    h = _resblock(h, tr3a_r[...], br3a_r[...], wr1a_r[...], br1a_r[...])
    h = _resblock(h, tr3b_r[...], br3b_r[...], wr1b_r[...], br1b_r[...])
    z = h
    z_o[...] = z

    cbm = cbm_r[...]
    c2 = jnp.sum(cbm * cbm, axis=0, keepdims=True)
    dist = c2 - 2.0 * _mm(z, cbm)
    mn = jnp.min(dist, axis=1, keepdims=True)
    ii = jax.lax.broadcasted_iota(jnp.int32, (N32, N32), 1)
    idx_o[...] = jnp.min(jnp.where(dist == mn, ii, jnp.int32(1 << 30)),
                         axis=1, keepdims=True)


def _dec_body(zq_r, td0_r, bd0_r,
              dr3a_r, dbr3a_r, dw1a_r, dbr1a_r,
              dr3b_r, dbr3b_r, dw1b_r, dbr1b_r,
              tt1_r, tb1_r, tt2_r, tb2_r, xp_o):
    relu = jax.nn.relu
    zq = zq_r[...]
    h = _conv3x3(zq, td0_r[...], bd0_r[...], _mmb)
    h = _resblock(h, dr3a_r[...], dbr3a_r[...], dw1a_r[...], dbr1a_r[...], _mmb)
    h = _resblock(h, dr3b_r[...], dbr3b_r[...], dw1b_r[...], dbr1b_r[...], _mmb)

    hp = jnp.pad(h.reshape(LS, LS, HID), ((1, 1), (1, 1), (0, 0)))
    tt1 = tt1_r[...]
    G = [[None, None], [None, None]]
    for s in (0, 1):
        for sx in (0, 1):
            a = None
            for dy in (s, s + 2):
                a0 = (s - 2 + dy) // 2 + 1
                for dx in (sx, sx + 2):
                    b0 = (sx - 2 + dx) // 2 + 1
                    sl = hp[a0:a0 + LS, b0:b0 + LS, :].reshape(N32, HID)
                    t = _mmb(sl, tt1[dy * 4 + dx])
                    a = t if a is None else a + t
            g = relu(a + tb1_r[...])
            G[s][sx] = jnp.pad(g.reshape(LS, LS, HID // 2),
                               ((1, 1), (1, 1), (0, 0)))
    tt2 = tt2_r[...]
    outs = []
    for ry in range(4):
        for rx in range(4):
            a = None
            for dy in (ry % 2, ry % 2 + 2):
                m = (ry - 2 + dy) // 2
                s = m % 2
                a0 = 1 + (m - s) // 2
                for dx in (rx % 2, rx % 2 + 2):
                    mx = (rx - 2 + dx) // 2
                    sx = mx % 2
                    b0 = 1 + (mx - sx) // 2
                    sl = G[s][sx][a0:a0 + LS, b0:b0 + LS, :].reshape(N32, HID // 2)
                    t = _mmb(sl, tt2[dy * 4 + dx])
                    a = t if a is None else a + t
            outs.append(a + tb2_r[...])
    xp_o[...] = jnp.concatenate(outs, axis=1)


def _sc_gather(table, idx):
    info = plsc.get_sparse_core_info()
    NC, NS = info.num_cores, info.num_subcores
    NW = NC * NS
    b_per_w = N32 // NW
    mesh = plsc.VectorSubcoreMesh(core_axis_name="c", subcore_axis_name="s")

    @functools.partial(
        pl.kernel, mesh=mesh,
        out_type=jax.ShapeDtypeStruct((N32, HID), jnp.float32),
        scratch_types=[
            pltpu.VMEM((b_per_w,), jnp.int32),
            pltpu.VMEM((b_per_w, HID), jnp.float32),
            pltpu.SemaphoreType.DMA,
        ],
    )
    def k(table_hbm, idx_hbm, out_hbm, idx_v, rows_v, sem):
        wid = lax.axis_index("s") * NC + lax.axis_index("c")
        base = wid * b_per_w
        pltpu.sync_copy(idx_hbm.at[pl.ds(base, b_per_w)], idx_v)
        pltpu.async_copy(table_hbm.at[idx_v], rows_v, sem).wait()
        pltpu.sync_copy(rows_v, out_hbm.at[pl.ds(base, b_per_w)])

    return k(table, idx)


def kernel(x, code_books, params):
    p = params
    f32 = jnp.float32

    x_cl = x.transpose(1, 2, 0)
    xpad = jnp.pad(x_cl, ((1, 1), (1, 1), (0, 0)))
    span = 4 * (LS - 1) + 2
    blocks = []
    for q in (0, 1):
        for qx in (0, 1):
            patch = jnp.concatenate(
                [xpad[2 * q + dy:2 * q + dy + span:4,
                      2 * qx + dx:2 * qx + dx + span:4, :]
                 for dy in range(4) for dx in range(4)], axis=-1)
            blocks.append(patch.reshape(N32, 48))
    p1 = jnp.concatenate(blocks, axis=0)

    def taps(w, k):
        return w.transpose(2, 3, 1, 0).reshape(k * k, w.shape[1], w.shape[0])

    w1m = p['enc_w1'].transpose(2, 3, 1, 0).reshape(48, HID // 2)
    row = lambda b: b.reshape(1, -1)

    enc_args = [
        p1, w1m, row(p['enc_b1']),
        taps(p['enc_w2'], 4), row(p['enc_b2']),
        taps(p['enc_w3'], 3), row(p['enc_b3']),
        taps(p['enc_res_w3_0'], 3), row(p['enc_res_b3_0']),
        p['enc_res_w1_0'][:, :, 0, 0].T, row(p['enc_res_b1_0']),
        taps(p['enc_res_w3_1'], 3), row(p['enc_res_b3_1']),
        p['enc_res_w1_1'][:, :, 0, 0].T, row(p['enc_res_b1_1']),
        code_books.reshape(HID, N32),
    ]
    zf, idx2d = pl.pallas_call(
        _enc_body,
        out_shape=(jax.ShapeDtypeStruct((N32, HID), f32),
                   jax.ShapeDtypeStruct((N32, 1), jnp.int32)),
    )(*enc_args)

    cbt = code_books.reshape(HID, N32).T
    zqf = _sc_gather(cbt, idx2d.reshape(N32))

    dec_args = [
        zqf,
        taps(p['dec_w0'], 3), row(p['dec_b0']),
        taps(p['dec_res_w3_0'], 3), row(p['dec_res_b3_0']),
        p['dec_res_w1_0'][:, :, 0, 0].T, row(p['dec_res_b1_0']),
        taps(p['dec_res_w3_1'], 3), row(p['dec_res_b3_1']),
        p['dec_res_w1_1'][:, :, 0, 0].T, row(p['dec_res_b1_1']),
        taps(p['dec_tw1'], 4), row(p['dec_tb1']),
        taps(p['dec_tw2'], 4), row(p['dec_tb2']),
    ]
    xp = pl.pallas_call(
        _dec_body,
        out_shape=jax.ShapeDtypeStruct((N32, 48), f32),
    )(*dec_args)

    x_pred = (xp.reshape(LS, LS, 4, 4, 3)
                .transpose(4, 0, 2, 1, 3).reshape(3, INP, INP))
    z_st = zf.reshape(LS, LS, HID).transpose(2, 0, 1)
    zq = zqf.reshape(LS, LS, HID).transpose(2, 0, 1)
    return (x_pred, z_st, zq)
